# baseline (speedup 1.0000x reference)
import functools

import jax
import jax.numpy as jnp
from jax.experimental import pallas as pl
from jax.experimental.pallas import tpu as pltpu

_SPARSITY_FRAC = 0.05


def _prefix2(x, S):
    ii = jax.lax.broadcasted_iota(jnp.int32, (128, 128), 0)
    jj = jax.lax.broadcasted_iota(jnp.int32, (128, 128), 1)
    lt128 = (ii < jj).astype(jnp.float32)
    lt32 = (ii[:32, :32] < jj[:32, :32]).astype(jnp.float32)
    std = (((1,), (0,)), ((), ()))
    withins = []
    tots = []
    for c in range(S // 128):
        xc = x[:, c * 128:(c + 1) * 128]
        w = jax.lax.dot_general(xc, lt128, std,
                                preferred_element_type=jnp.float32)
        withins.append(w)
        tots.append(w[:, 127:128] + xc[:, 127:128])
    tot = jnp.concatenate(tots, axis=1)
    cpref = jax.lax.dot_general(tot, lt32, std,
                                preferred_element_type=jnp.float32,
                                precision=jax.lax.Precision.HIGHEST)
    return jnp.concatenate(
        [withins[c] + cpref[:, c:c + 1] for c in range(S // 128)], axis=1)


def _order_tail(k, S, P, key_b, sel_b, c_b):
    p_col = jax.lax.broadcasted_iota(jnp.int32, (P, 1), 0).astype(jnp.float32)
    onehot = jnp.logical_and(c_b == p_col, sel_b)
    j_row = jax.lax.broadcasted_iota(jnp.int32, (1, S), 1).astype(jnp.float32)
    hi_row = jax.lax.shift_right_arithmetic(key_b, 16).astype(jnp.float32)
    lo_row = (key_b & jnp.int32(0xFFFF)).astype(jnp.float32)
    ohf = onehot.astype(jnp.float32)
    cand_idx = jnp.sum(ohf * j_row, axis=1, keepdims=True)
    cand_hi = jnp.sum(ohf * hi_row, axis=1, keepdims=True)
    cand_lo = jnp.sum(ohf * lo_row, axis=1, keepdims=True)

    ee = jax.lax.broadcasted_iota(jnp.int32, (P, P), 0)
    ff = jax.lax.broadcasted_iota(jnp.int32, (P, P), 1)
    eye = (ee == ff).astype(jnp.float32)
    tdims = (((0,), (0,)), ((), ()))
    hp = jax.lax.Precision.HIGHEST
    cand_idx_r = jax.lax.dot_general(cand_idx, eye, tdims,
                                     preferred_element_type=jnp.float32,
                                     precision=hp)
    cand_hi_r = jax.lax.dot_general(cand_hi, eye, tdims,
                                    preferred_element_type=jnp.float32,
                                    precision=hp)
    cand_lo_r = jax.lax.dot_general(cand_lo, eye, tdims,
                                    preferred_element_type=jnp.float32,
                                    precision=hp)

    valid_c = p_col < k
    valid_r = jax.lax.broadcasted_iota(jnp.int32, (1, P), 1) < k
    ahead = jnp.logical_or(
        cand_hi > cand_hi_r,
        jnp.logical_and(
            cand_hi == cand_hi_r,
            jnp.logical_or(
                cand_lo > cand_lo_r,
                jnp.logical_and(cand_lo == cand_lo_r, cand_idx < cand_idx_r),
            ),
        ),
    )
    ahead = jnp.logical_and(ahead, jnp.logical_and(valid_c, valid_r))
    rank_r = jnp.sum(ahead.astype(jnp.float32), axis=0, keepdims=True)
    rank_r = jnp.where(valid_r, rank_r, jnp.float32(1e9))

    hit = (rank_r == p_col).astype(jnp.float32)
    return jnp.sum(hit * cand_idx_r, axis=1, keepdims=True).astype(jnp.int32)


def _topk2(k, S, P, s, mask_ref, tidx_ref):
    bits = jax.lax.bitcast_convert_type(s, jnp.int32)
    key = bits ^ (jax.lax.shift_right_arithmetic(bits, 31) & jnp.int32(0x7FFFFFFF))

    kvec = jnp.full((2, 1), k, jnp.int32)
    nonneg = (key >= 0).astype(jnp.int32)
    cnt0 = jnp.sum(nonneg, axis=1, keepdims=True)
    take = (kvec <= cnt0).astype(jnp.int32)
    active = take * nonneg + (1 - take) * (1 - nonneg)
    kk = kvec - (1 - take) * cnt0
    T = (1 - take) * jnp.int32(-2147483648)

    def bit_body(bi, carry):
        active, kk, T = carry
        b = 30 - bi
        bitset = jax.lax.shift_right_arithmetic(key, b) & 1
        hi = active * bitset
        cnt = jnp.sum(hi, axis=1, keepdims=True)
        take = (kk <= cnt).astype(jnp.int32)
        active = take * hi + (1 - take) * active * (1 - bitset)
        kk = kk - (1 - take) * cnt
        T = T | (take * jax.lax.shift_left(jnp.int32(1), b))
        return active, kk, T

    _, _, T = jax.lax.fori_loop(0, 31, bit_body, (active, kk, T))

    gt = key > T
    eq = key == T
    ngt = jnp.sum(gt.astype(jnp.int32), axis=1, keepdims=True)
    m = (kvec - ngt).astype(jnp.float32)
    tie_pref = _prefix2(eq.astype(jnp.float32), S)
    sel = jnp.logical_or(gt, jnp.logical_and(eq, tie_pref < m))
    mask_ref[...] = jnp.where(sel, jnp.float32(1.0 / k), jnp.float32(0.0))

    c_row = _prefix2(sel.astype(jnp.float32), S)
    tidx_ref[0] = _order_tail(k, S, P, key[0:1], sel[0:1], c_row[0:1])
    tidx_ref[1] = _order_tail(k, S, P, key[1:2], sel[1:2], c_row[1:2])


def _mega_kernel(k, S, D, F, NA, MB, NC, R, P,
                 emb_ref, w1_ref, b1_ref, w2_ref,
                 attn_ref, tidx_ref, sc_ref, mask_ref):
    t = pl.program_id(0)

    def scores_to(row, m):
        e = emb_ref[0].astype(jnp.bfloat16)
        h = jnp.maximum(
            jnp.dot(e, w1_ref[...], preferred_element_type=jnp.float32)
            + b1_ref[...],
            0.0,
        )
        hb = h.astype(jnp.bfloat16)
        s_row = jax.lax.dot_general(
            w2_ref[...], hb, (((1,), (1,)), ((), ())),
            preferred_element_type=jnp.float32)
        sc_ref[row:row + 1, pl.ds(m * MB, MB)] = s_row

    @pl.when(t < NA)
    def _scores_b0():
        scores_to(0, t)

    @pl.when(jnp.logical_and(t >= NA, t < 2 * NA))
    def _scores_b1():
        scores_to(1, t - NA)

    @pl.when(t == 2 * NA)
    def _topk_phase():
        _topk2(k, S, P, sc_ref[...], mask_ref, tidx_ref)

    @pl.when(jnp.logical_and(t > 2 * NA, t <= 2 * NA + NC))
    def _bcast_b0():
        attn_ref[...] = jnp.broadcast_to(mask_ref[0:1][None], (1, R, S))

    @pl.when(t > 2 * NA + NC)
    def _bcast_b1():
        attn_ref[...] = jnp.broadcast_to(mask_ref[1:2][None], (1, R, S))


def kernel(embeddings, W1, b1, W2, b2):
    B, S, D = embeddings.shape
    F = W1.shape[1]
    k = max(1, int(S * _SPARSITY_FRAC))

    MB = 4096
    NA = S // MB
    R = 512
    NC = S // R
    P = 256
    na, nc = NA, NC

    attn, tidx = pl.pallas_call(
        functools.partial(_mega_kernel, k, S, D, F, NA, MB, NC, R, P),
        grid=(2 * NA + 1 + 2 * NC,),
        in_specs=[
            pl.BlockSpec((1, MB, D),
                         lambda t: (jnp.minimum(t, 2 * na - 1) // na,
                                    jnp.minimum(t, 2 * na - 1) % na, 0)),
            pl.BlockSpec((D, F), lambda t: (0, 0)),
            pl.BlockSpec((1, F), lambda t: (0, 0)),
            pl.BlockSpec((1, F), lambda t: (0, 0)),
        ],
        out_specs=[
            pl.BlockSpec((1, R, S),
                         lambda t: (jnp.maximum(t - (2 * na + 1), 0) // nc,
                                    jnp.maximum(t - (2 * na + 1), 0) % nc, 0)),
            pl.BlockSpec((2, P, 1), lambda t: (0, 0, 0)),
        ],
        out_shape=[
            jax.ShapeDtypeStruct((B, S, S), jnp.float32),
            jax.ShapeDtypeStruct((B, P, 1), jnp.int32),
        ],
        scratch_shapes=[
            pltpu.VMEM((2, S), jnp.float32),
            pltpu.VMEM((2, S), jnp.float32),
        ],
    )(embeddings, W1.astype(jnp.bfloat16), b1.reshape(1, F),
      W2.reshape(1, F).astype(jnp.bfloat16))

    top_indices = tidx[:, :k, 0]
    return attn, top_indices


# speedup vs baseline: 1.0170x; 1.0170x over previous
import functools

import jax
import jax.numpy as jnp
from jax.experimental import pallas as pl
from jax.experimental.pallas import tpu as pltpu

_SPARSITY_FRAC = 0.05


def _prefix2(x, S):
    ii = jax.lax.broadcasted_iota(jnp.int32, (128, 128), 0)
    jj = jax.lax.broadcasted_iota(jnp.int32, (128, 128), 1)
    lt128 = (ii < jj).astype(jnp.float32)
    lt32 = (ii[:32, :32] < jj[:32, :32]).astype(jnp.float32)
    std = (((1,), (0,)), ((), ()))
    withins = []
    tots = []
    for c in range(S // 128):
        xc = x[:, c * 128:(c + 1) * 128]
        w = jax.lax.dot_general(xc, lt128, std,
                                preferred_element_type=jnp.float32)
        withins.append(w)
        tots.append(w[:, 127:128] + xc[:, 127:128])
    tot = jnp.concatenate(tots, axis=1)
    cpref = jax.lax.dot_general(tot, lt32, std,
                                preferred_element_type=jnp.float32,
                                precision=jax.lax.Precision.HIGHEST)
    return jnp.concatenate(
        [withins[c] + cpref[:, c:c + 1] for c in range(S // 128)], axis=1)


def _order_tail(k, S, P, key_b, sel_b, c_b):
    p_col = jax.lax.broadcasted_iota(jnp.int32, (P, 1), 0).astype(jnp.float32)
    onehot = jnp.logical_and(c_b == p_col, sel_b)
    j_row = jax.lax.broadcasted_iota(jnp.int32, (1, S), 1).astype(jnp.float32)
    hi_row = jax.lax.shift_right_arithmetic(key_b, 16).astype(jnp.float32)
    lo_row = (key_b & jnp.int32(0xFFFF)).astype(jnp.float32)
    ohf = onehot.astype(jnp.float32)
    cand_idx = jnp.sum(ohf * j_row, axis=1, keepdims=True)
    cand_hi = jnp.sum(ohf * hi_row, axis=1, keepdims=True)
    cand_lo = jnp.sum(ohf * lo_row, axis=1, keepdims=True)

    ee = jax.lax.broadcasted_iota(jnp.int32, (P, P), 0)
    ff = jax.lax.broadcasted_iota(jnp.int32, (P, P), 1)
    eye = (ee == ff).astype(jnp.float32)
    tdims = (((0,), (0,)), ((), ()))
    hp = jax.lax.Precision.HIGHEST
    cand_idx_r = jax.lax.dot_general(cand_idx, eye, tdims,
                                     preferred_element_type=jnp.float32,
                                     precision=hp)
    cand_hi_r = jax.lax.dot_general(cand_hi, eye, tdims,
                                    preferred_element_type=jnp.float32,
                                    precision=hp)
    cand_lo_r = jax.lax.dot_general(cand_lo, eye, tdims,
                                    preferred_element_type=jnp.float32,
                                    precision=hp)

    valid_c = p_col < k
    valid_r = jax.lax.broadcasted_iota(jnp.int32, (1, P), 1) < k
    ahead = jnp.logical_or(
        cand_hi > cand_hi_r,
        jnp.logical_and(
            cand_hi == cand_hi_r,
            jnp.logical_or(
                cand_lo > cand_lo_r,
                jnp.logical_and(cand_lo == cand_lo_r, cand_idx < cand_idx_r),
            ),
        ),
    )
    ahead = jnp.logical_and(ahead, jnp.logical_and(valid_c, valid_r))
    rank_r = jnp.sum(ahead.astype(jnp.float32), axis=0, keepdims=True)
    rank_r = jnp.where(valid_r, rank_r, jnp.float32(1e9))

    hit = (rank_r == p_col).astype(jnp.float32)
    return jnp.sum(hit * cand_idx_r, axis=1, keepdims=True).astype(jnp.int32)


def _topk2(k, S, P, s, mask_ref, tidx_ref):
    bits = jax.lax.bitcast_convert_type(s, jnp.int32)
    key = bits ^ (jax.lax.shift_right_arithmetic(bits, 31) & jnp.int32(0x7FFFFFFF))

    kvec = jnp.full((2, 1), k, jnp.int32)
    nonneg = (key >= 0).astype(jnp.int32)
    cnt0 = jnp.sum(nonneg, axis=1, keepdims=True)
    take = (kvec <= cnt0).astype(jnp.int32)
    active = take * nonneg + (1 - take) * (1 - nonneg)
    kk = kvec - (1 - take) * cnt0
    T = (1 - take) * jnp.int32(-2147483648)

    def bit_body(bi, carry):
        active, kk, T = carry
        b = 30 - bi
        bitset = jax.lax.shift_right_arithmetic(key, b) & 1
        hi = active * bitset
        cnt = jnp.sum(hi, axis=1, keepdims=True)
        take = (kk <= cnt).astype(jnp.int32)
        active = take * hi + (1 - take) * active * (1 - bitset)
        kk = kk - (1 - take) * cnt
        T = T | (take * jax.lax.shift_left(jnp.int32(1), b))
        return active, kk, T

    _, _, T = jax.lax.fori_loop(0, 31, bit_body, (active, kk, T))

    gt = key > T
    eq = key == T
    ngt = jnp.sum(gt.astype(jnp.int32), axis=1, keepdims=True)
    m = (kvec - ngt).astype(jnp.float32)
    tie_pref = _prefix2(eq.astype(jnp.float32), S)
    sel = jnp.logical_or(gt, jnp.logical_and(eq, tie_pref < m))
    mask_ref[...] = jnp.where(sel, jnp.float32(1.0 / k), jnp.float32(0.0))

    c_row = _prefix2(sel.astype(jnp.float32), S)
    tidx_ref[0] = _order_tail(k, S, P, key[0:1], sel[0:1], c_row[0:1])
    tidx_ref[1] = _order_tail(k, S, P, key[1:2], sel[1:2], c_row[1:2])


def _mega_kernel(k, S, D, F, NA, MB, NC, R, P,
                 emb_ref, w1_ref, b1_ref, w2_ref,
                 attn_ref, tidx_ref, sc_ref, mask_ref):
    t = pl.program_id(0)

    def scores_to(row, m):
        e = emb_ref[0].astype(jnp.bfloat16)
        h = jnp.maximum(
            jnp.dot(e, w1_ref[...], preferred_element_type=jnp.float32)
            + b1_ref[...],
            0.0,
        )
        hb = h.astype(jnp.bfloat16)
        s_row = jax.lax.dot_general(
            w2_ref[...], hb, (((1,), (1,)), ((), ())),
            preferred_element_type=jnp.float32)
        sc_ref[row:row + 1, pl.ds(m * MB, MB)] = s_row

    @pl.when(t < NA)
    def _scores_b0():
        scores_to(0, t)

    @pl.when(jnp.logical_and(t >= NA, t < 2 * NA))
    def _scores_b1():
        scores_to(1, t - NA)

    @pl.when(t == 2 * NA)
    def _topk_phase():
        _topk2(k, S, P, sc_ref[...], mask_ref, tidx_ref)

    @pl.when(jnp.logical_and(t > 2 * NA, t <= 2 * NA + NC))
    def _bcast_b0():
        attn_ref[...] = jnp.broadcast_to(mask_ref[0:1][None], (1, R, S))

    @pl.when(t > 2 * NA + NC)
    def _bcast_b1():
        attn_ref[...] = jnp.broadcast_to(mask_ref[1:2][None], (1, R, S))


def kernel(embeddings, W1, b1, W2, b2):
    B, S, D = embeddings.shape
    F = W1.shape[1]
    k = max(1, int(S * _SPARSITY_FRAC))

    MB = 2048
    NA = S // MB
    R = 512
    NC = S // R
    P = 256
    na, nc = NA, NC

    attn, tidx = pl.pallas_call(
        functools.partial(_mega_kernel, k, S, D, F, NA, MB, NC, R, P),
        grid=(2 * NA + 1 + 2 * NC,),
        in_specs=[
            pl.BlockSpec((1, MB, D),
                         lambda t: (jnp.minimum(t, 2 * na - 1) // na,
                                    jnp.minimum(t, 2 * na - 1) % na, 0)),
            pl.BlockSpec((D, F), lambda t: (0, 0)),
            pl.BlockSpec((1, F), lambda t: (0, 0)),
            pl.BlockSpec((1, F), lambda t: (0, 0)),
        ],
        out_specs=[
            pl.BlockSpec((1, R, S),
                         lambda t: (jnp.maximum(t - (2 * na + 1), 0) // nc,
                                    jnp.maximum(t - (2 * na + 1), 0) % nc, 0)),
            pl.BlockSpec((2, P, 1), lambda t: (0, 0, 0)),
        ],
        out_shape=[
            jax.ShapeDtypeStruct((B, S, S), jnp.float32),
            jax.ShapeDtypeStruct((B, P, 1), jnp.int32),
        ],
        scratch_shapes=[
            pltpu.VMEM((2, S), jnp.float32),
            pltpu.VMEM((2, S), jnp.float32),
        ],
    )(embeddings, W1.astype(jnp.bfloat16), b1.reshape(1, F),
      W2.reshape(1, F).astype(jnp.bfloat16))

    top_indices = tidx[:, :k, 0]
    return attn, top_indices
